# Initial kernel scaffold; baseline (speedup 1.0000x reference)
import jax
import jax.numpy as jnp
from jax.experimental import pallas as pl


def kernel(x, W, keys):
    raise NotImplementedError("write your pallas kernel here")


# speedup vs baseline: 3.2657x; 3.2657x over previous
import functools

import jax
import jax.numpy as jnp
from jax.experimental import pallas as pl

DIM = 2048
KNOWLEDGE_DIM = 512
HALF = KNOWLEDGE_DIM // 2
NUM_KEYS = 1024
NUM_CANDIDATES = 32


def _topk32(s, iota, n):
    vals, idxs = [], []
    for _ in range(NUM_CANDIDATES):
        m = jnp.max(s, axis=1, keepdims=True)
        pos = jnp.min(jnp.where(s == m, iota, n), axis=1, keepdims=True)
        vals.append(m)
        idxs.append(pos)
        s = jnp.where(iota == pos, -jnp.inf, s)
    return vals, idxs


def _body(x_ref, w_ref, keys_ref, idx_ref, scr_ref):
    r = x_ref.shape[0]
    q = jnp.dot(x_ref[...], w_ref[...], preferred_element_type=jnp.float32)
    keys = keys_ref[...]
    s1 = jnp.dot(q[:, :HALF], keys[0], preferred_element_type=jnp.float32)
    s2 = jnp.dot(q[:, HALF:], keys[1], preferred_element_type=jnp.float32)

    iota_k = jax.lax.broadcasted_iota(jnp.int32, (r, NUM_KEYS), 1)
    v1, i1 = _topk32(s1, iota_k, NUM_KEYS)
    v2, i2 = _topk32(s2, iota_k, NUM_KEYS)
    v2c = jnp.concatenate(v2, axis=1)
    i2c = jnp.concatenate(i2, axis=1)

    comb_s = jnp.concatenate([v1[t] + v2c for t in range(NUM_CANDIDATES)], axis=1)
    comb_i = jnp.concatenate(
        [i1[t] * NUM_KEYS + i2c for t in range(NUM_CANDIDATES)], axis=1)

    vals, idxs = [], []
    s = comb_s
    for _ in range(NUM_CANDIDATES):
        m = jnp.max(s, axis=1, keepdims=True)
        pos = jnp.min(jnp.where(s == m, iota_k, NUM_KEYS), axis=1, keepdims=True)
        hit = iota_k == pos
        vals.append(m)
        idxs.append(jnp.max(jnp.where(hit, comb_i, -1), axis=1, keepdims=True))
        s = jnp.where(hit, -jnp.inf, s)

    top_s = jnp.concatenate(vals, axis=1)
    top_i = jnp.concatenate(idxs, axis=1)

    mx = jnp.max(top_s, axis=1, keepdims=True)
    e = jnp.exp(top_s - mx)
    p = e / jnp.sum(e, axis=1, keepdims=True)

    idx_ref[...] = top_i
    scr_ref[...] = p


@jax.jit
def kernel(x, W, keys):
    bsz, seq_len, d = x.shape
    n = bsz * seq_len
    xf = x.reshape(n, d)
    keys_t = jnp.transpose(keys, (0, 2, 1))

    r = 256 if n % 256 == 0 else n
    grid = n // r

    idx, scr = pl.pallas_call(
        _body,
        grid=(grid,),
        in_specs=[
            pl.BlockSpec((r, d), lambda i: (i, 0)),
            pl.BlockSpec((d, KNOWLEDGE_DIM), lambda i: (0, 0)),
            pl.BlockSpec((2, HALF, NUM_KEYS), lambda i: (0, 0, 0)),
        ],
        out_specs=[
            pl.BlockSpec((r, NUM_CANDIDATES), lambda i: (i, 0)),
            pl.BlockSpec((r, NUM_CANDIDATES), lambda i: (i, 0)),
        ],
        out_shape=[
            jax.ShapeDtypeStruct((n, NUM_CANDIDATES), jnp.int32),
            jax.ShapeDtypeStruct((n, NUM_CANDIDATES), jnp.float32),
        ],
    )(xf, W, keys_t)

    return (idx.reshape(bsz, seq_len, NUM_CANDIDATES),
            scr.reshape(bsz, seq_len, NUM_CANDIDATES))


# speedup vs baseline: 3.9392x; 1.2062x over previous
import functools

import jax
import jax.numpy as jnp
from jax.experimental import pallas as pl

DIM = 2048
KNOWLEDGE_DIM = 512
HALF = KNOWLEDGE_DIM // 2
NUM_KEYS = 1024
NUM_CANDIDATES = 32


def _topk32(s, iota, n):
    vals, idxs = [], []
    for _ in range(NUM_CANDIDATES):
        m = jnp.max(s, axis=1, keepdims=True)
        pos = jnp.min(jnp.where(s == m, iota, n), axis=1, keepdims=True)
        vals.append(m)
        idxs.append(pos)
        s = jnp.where(iota == pos, -jnp.inf, s)
    return vals, idxs


def _body(x_ref, w_ref, keys_ref, idx_ref, scr_ref):
    r = x_ref.shape[0]
    q = jnp.dot(x_ref[...], w_ref[...], preferred_element_type=jnp.float32)
    keys = keys_ref[...]
    s1 = jnp.dot(q[:, :HALF], keys[0], preferred_element_type=jnp.float32)
    s2 = jnp.dot(q[:, HALF:], keys[1], preferred_element_type=jnp.float32)

    iota_k = jax.lax.broadcasted_iota(jnp.int32, (r, NUM_KEYS), 1)
    v1, i1 = _topk32(s1, iota_k, NUM_KEYS)
    v2, i2 = _topk32(s2, iota_k, NUM_KEYS)
    v2c = jnp.concatenate(v2, axis=1)
    i2c = jnp.concatenate(i2, axis=1)

    comb_s, comb_i = [], []
    ncand = 0
    for t in range(NUM_CANDIDATES):
        c = NUM_CANDIDATES // (t + 1)
        comb_s.append(v1[t] + v2c[:, :c])
        comb_i.append(i1[t] * NUM_KEYS + i2c[:, :c])
        ncand += c
    npad = 128 - ncand
    comb_s.append(jnp.full((r, npad), -jnp.inf, jnp.float32))
    comb_i.append(jnp.zeros((r, npad), jnp.int32))
    comb_s = jnp.concatenate(comb_s, axis=1)
    comb_i = jnp.concatenate(comb_i, axis=1)

    iota_c = jax.lax.broadcasted_iota(jnp.int32, (r, 128), 1)
    vals, idxs = [], []
    s = comb_s
    for _ in range(NUM_CANDIDATES):
        m = jnp.max(s, axis=1, keepdims=True)
        pos = jnp.min(jnp.where(s == m, iota_c, 128), axis=1, keepdims=True)
        hit = iota_c == pos
        vals.append(m)
        idxs.append(jnp.max(jnp.where(hit, comb_i, -1), axis=1, keepdims=True))
        s = jnp.where(hit, -jnp.inf, s)

    top_s = jnp.concatenate(vals, axis=1)
    top_i = jnp.concatenate(idxs, axis=1)

    mx = jnp.max(top_s, axis=1, keepdims=True)
    e = jnp.exp(top_s - mx)
    p = e / jnp.sum(e, axis=1, keepdims=True)

    idx_ref[...] = top_i
    scr_ref[...] = p


@jax.jit
def kernel(x, W, keys):
    bsz, seq_len, d = x.shape
    n = bsz * seq_len
    xf = x.reshape(n, d)
    keys_t = jnp.transpose(keys, (0, 2, 1))

    r = 256 if n % 256 == 0 else n
    grid = n // r

    idx, scr = pl.pallas_call(
        _body,
        grid=(grid,),
        in_specs=[
            pl.BlockSpec((r, d), lambda i: (i, 0)),
            pl.BlockSpec((d, KNOWLEDGE_DIM), lambda i: (0, 0)),
            pl.BlockSpec((2, HALF, NUM_KEYS), lambda i: (0, 0, 0)),
        ],
        out_specs=[
            pl.BlockSpec((r, NUM_CANDIDATES), lambda i: (i, 0)),
            pl.BlockSpec((r, NUM_CANDIDATES), lambda i: (i, 0)),
        ],
        out_shape=[
            jax.ShapeDtypeStruct((n, NUM_CANDIDATES), jnp.int32),
            jax.ShapeDtypeStruct((n, NUM_CANDIDATES), jnp.float32),
        ],
    )(xf, W, keys_t)

    return (idx.reshape(bsz, seq_len, NUM_CANDIDATES),
            scr.reshape(bsz, seq_len, NUM_CANDIDATES))
